# baseline (speedup 1.0000x reference)
import functools

import jax
import jax.numpy as jnp
from jax import lax
from jax.experimental import pallas as pl
from jax.experimental.pallas import tpu as pltpu
from jax.experimental.pallas import tpu_sc as plsc

_NUM_EMB = 10
_VOCAB = 100000
_EMB_DIM = 32
_N_NUM = 8
_T_IN = 168
_T_FC = 24
_T = _T_IN + _T_FC
_B = 1024
_EMB_FEAT = _NUM_EMB * _EMB_DIM
_FEAT = _N_NUM + _EMB_FEAT

_NC, _NS = 2, 16
_NW = _NC * _NS
_ROWS = _B * _NUM_EMB
_ROWS_PER_W = _ROWS // _NW
_CHUNK = 64
_NCHUNK = _ROWS_PER_W // _CHUNK


def _sc_gather_body(table_hbm, idx_hbm, out_hbm, idx_v, rows_v, sem):
    wid = lax.axis_index("s") * _NC + lax.axis_index("c")
    pltpu.sync_copy(idx_hbm.at[wid], idx_v)
    copies = [
        pltpu.async_copy(table_hbm.at[idx_v.at[j]], rows_v.at[j], sem)
        for j in range(_NCHUNK)
    ]
    for c in copies:
        c.wait()
    pltpu.sync_copy(rows_v, out_hbm.at[wid])


@functools.cache
def _make_sc_gather():
    return pl.kernel(
        _sc_gather_body,
        out_type=jax.ShapeDtypeStruct((_NW, _NCHUNK, _CHUNK, _EMB_DIM),
                                      jnp.float32),
        mesh=plsc.VectorSubcoreMesh(core_axis_name="c", subcore_axis_name="s",
                                    num_cores=_NC, num_subcores=_NS),
        scratch_types=[
            pltpu.VMEM((_NCHUNK, _CHUNK), jnp.int32),
            pltpu.VMEM((_NCHUNK, _CHUNK, _EMB_DIM), jnp.float32),
            pltpu.SemaphoreType.DMA,
        ],
        compiler_params=pltpu.CompilerParams(use_tc_tiling_on_sc=False),
    )

_BB = 8
_TB = 24
_NTB = _T // _TB
_NPB = _T_IN // _TB


def _asm_body(in_ref, emb_ref, past_ref, fut_ref):
    j = pl.program_id(1)
    num = in_ref[...][:, :, :_N_NUM].astype(jnp.float32)
    emb = emb_ref[...]
    full = jnp.concatenate(
        [num, jnp.broadcast_to(emb[:, None, :], (_BB, _TB, _EMB_FEAT))], axis=2)

    @pl.when(j < _NPB)
    def _():
        past_ref[...] = full

    @pl.when(j == _NPB)
    def _():
        fut_ref[...] = full


def _assemble(input, emb):
    return pl.pallas_call(
        _asm_body,
        grid=(_B // _BB, _NTB),
        in_specs=[
            pl.BlockSpec((_BB, _TB, _N_NUM + _NUM_EMB), lambda i, j: (i, j, 0)),
            pl.BlockSpec((_BB, _EMB_FEAT), lambda i, j: (i, 0)),
        ],
        out_specs=[
            pl.BlockSpec((_BB, _TB, _FEAT),
                         lambda i, j: (i, jnp.minimum(j, _NPB - 1), 0)),
            pl.BlockSpec((_BB, _TB, _FEAT), lambda i, j: (i, 0, 0)),
        ],
        out_shape=[
            jax.ShapeDtypeStruct((_B, _T_IN, _FEAT), jnp.float32),
            jax.ShapeDtypeStruct((_B, _T_FC, _FEAT), jnp.float32),
        ],
    )(input, emb)


def kernel(input, tables):
    idx = input[:, 0, _N_NUM:]
    flat_idx = idx + (jnp.arange(_NUM_EMB, dtype=jnp.int32) * _VOCAB)[None, :]
    idx_hbm = flat_idx.reshape(_NW, _NCHUNK, _CHUNK)
    table_flat = tables.reshape(_NUM_EMB * _VOCAB, _EMB_DIM)
    emb = _make_sc_gather()(table_flat, idx_hbm).reshape(_B, _EMB_FEAT)
    past, fut = _assemble(input, emb)
    return (past, fut)


# speedup vs baseline: 1.3882x; 1.3882x over previous
import functools

import jax
import jax.numpy as jnp
from jax import lax
from jax.experimental import pallas as pl
from jax.experimental.pallas import tpu as pltpu
from jax.experimental.pallas import tpu_sc as plsc

_NUM_EMB = 10
_VOCAB = 100000
_EMB_DIM = 32
_N_NUM = 8
_T_IN = 168
_T_FC = 24
_T = _T_IN + _T_FC
_B = 1024
_EMB_FEAT = _NUM_EMB * _EMB_DIM
_FEAT = _N_NUM + _EMB_FEAT

_NC, _NS = 2, 16
_NW = _NC * _NS
_ROWS = _B * _NUM_EMB
_ROWS_PER_W = _ROWS // _NW
_CHUNK = 64
_NCHUNK = _ROWS_PER_W // _CHUNK


def _sc_gather_body(table_hbm, idx_hbm, out_hbm, idx_v, rows_v, sem):
    wid = lax.axis_index("s") * _NC + lax.axis_index("c")
    pltpu.sync_copy(idx_hbm.at[wid], idx_v)
    copies = [
        pltpu.async_copy(table_hbm.at[idx_v.at[j]], rows_v.at[j], sem)
        for j in range(_NCHUNK)
    ]
    for c in copies:
        c.wait()
    pltpu.sync_copy(rows_v, out_hbm.at[wid])


@functools.cache
def _make_sc_gather():
    return pl.kernel(
        _sc_gather_body,
        out_type=jax.ShapeDtypeStruct((_NW, _NCHUNK, _CHUNK, _EMB_DIM),
                                      jnp.float32),
        mesh=plsc.VectorSubcoreMesh(core_axis_name="c", subcore_axis_name="s",
                                    num_cores=_NC, num_subcores=_NS),
        scratch_types=[
            pltpu.VMEM((_NCHUNK, _CHUNK), jnp.int32),
            pltpu.VMEM((_NCHUNK, _CHUNK, _EMB_DIM), jnp.float32),
            pltpu.SemaphoreType.DMA,
        ],
        compiler_params=pltpu.CompilerParams(use_tc_tiling_on_sc=False),
    )

_BB = 32
_TB = 24
_NTB = _T // _TB
_NPB = _T_IN // _TB


def _asm_body(in_ref, emb_ref, past_ref, fut_ref):
    j = pl.program_id(1)
    num = in_ref[...][:, :, :_N_NUM].astype(jnp.float32)
    emb = emb_ref[...]
    full = jnp.concatenate(
        [num, jnp.broadcast_to(emb[:, None, :], (_BB, _TB, _EMB_FEAT))], axis=2)

    @pl.when(j < _NPB)
    def _():
        past_ref[...] = full

    @pl.when(j == _NPB)
    def _():
        fut_ref[...] = full


def _assemble(input, emb):
    return pl.pallas_call(
        _asm_body,
        grid=(_B // _BB, _NTB),
        in_specs=[
            pl.BlockSpec((_BB, _TB, _N_NUM + _NUM_EMB), lambda i, j: (i, j, 0)),
            pl.BlockSpec((_BB, _EMB_FEAT), lambda i, j: (i, 0)),
        ],
        out_specs=[
            pl.BlockSpec((_BB, _TB, _FEAT),
                         lambda i, j: (i, jnp.minimum(j, _NPB - 1), 0)),
            pl.BlockSpec((_BB, _TB, _FEAT), lambda i, j: (i, 0, 0)),
        ],
        out_shape=[
            jax.ShapeDtypeStruct((_B, _T_IN, _FEAT), jnp.float32),
            jax.ShapeDtypeStruct((_B, _T_FC, _FEAT), jnp.float32),
        ],
    )(input, emb)


def kernel(input, tables):
    idx = input[:, 0, _N_NUM:]
    flat_idx = idx + (jnp.arange(_NUM_EMB, dtype=jnp.int32) * _VOCAB)[None, :]
    idx_hbm = flat_idx.reshape(_NW, _NCHUNK, _CHUNK)
    table_flat = tables.reshape(_NUM_EMB * _VOCAB, _EMB_DIM)
    emb = _make_sc_gather()(table_flat, idx_hbm).reshape(_B, _EMB_FEAT)
    past, fut = _assemble(input, emb)
    return (past, fut)


# speedup vs baseline: 1.4833x; 1.0685x over previous
import functools

import jax
import jax.numpy as jnp
from jax import lax
from jax.experimental import pallas as pl
from jax.experimental.pallas import tpu as pltpu
from jax.experimental.pallas import tpu_sc as plsc

_NUM_EMB = 10
_VOCAB = 100000
_EMB_DIM = 32
_N_NUM = 8
_T_IN = 168
_T_FC = 24
_T = _T_IN + _T_FC
_B = 1024
_EMB_FEAT = _NUM_EMB * _EMB_DIM
_FEAT = _N_NUM + _EMB_FEAT

_NC, _NS = 2, 16
_NW = _NC * _NS
_ROWS = _B * _NUM_EMB
_ROWS_PER_W = _ROWS // _NW
_CHUNK = 64
_NCHUNK = _ROWS_PER_W // _CHUNK


def _sc_gather_body(table_hbm, idx_hbm, out_hbm, idx_v, rows_v, sem):
    wid = lax.axis_index("s") * _NC + lax.axis_index("c")
    pltpu.sync_copy(idx_hbm.at[wid], idx_v)
    copies = [
        pltpu.async_copy(table_hbm.at[idx_v.at[j]], rows_v.at[j], sem)
        for j in range(_NCHUNK)
    ]
    for c in copies:
        c.wait()
    pltpu.sync_copy(rows_v, out_hbm.at[wid])


@functools.cache
def _make_sc_gather():
    return pl.kernel(
        _sc_gather_body,
        out_type=jax.ShapeDtypeStruct((_NW, _NCHUNK, _CHUNK, _EMB_DIM),
                                      jnp.float32),
        mesh=plsc.VectorSubcoreMesh(core_axis_name="c", subcore_axis_name="s",
                                    num_cores=_NC, num_subcores=_NS),
        scratch_types=[
            pltpu.VMEM((_NCHUNK, _CHUNK), jnp.int32),
            pltpu.VMEM((_NCHUNK, _CHUNK, _EMB_DIM), jnp.float32),
            pltpu.SemaphoreType.DMA,
        ],
        compiler_params=pltpu.CompilerParams(use_tc_tiling_on_sc=False),
    )

_BB = 64
_TB = 24
_NTB = _T // _TB
_NPB = _T_IN // _TB


def _asm_body(in_ref, emb_ref, past_ref, fut_ref):
    j = pl.program_id(1)
    num = in_ref[...][:, :, :_N_NUM].astype(jnp.float32)
    emb = emb_ref[...]
    full = jnp.concatenate(
        [num, jnp.broadcast_to(emb[:, None, :], (_BB, _TB, _EMB_FEAT))], axis=2)

    @pl.when(j < _NPB)
    def _():
        past_ref[...] = full

    @pl.when(j == _NPB)
    def _():
        fut_ref[...] = full


def _assemble(input, emb):
    return pl.pallas_call(
        _asm_body,
        grid=(_B // _BB, _NTB),
        in_specs=[
            pl.BlockSpec((_BB, _TB, _N_NUM + _NUM_EMB), lambda i, j: (i, j, 0)),
            pl.BlockSpec((_BB, _EMB_FEAT), lambda i, j: (i, 0)),
        ],
        out_specs=[
            pl.BlockSpec((_BB, _TB, _FEAT),
                         lambda i, j: (i, jnp.minimum(j, _NPB - 1), 0)),
            pl.BlockSpec((_BB, _TB, _FEAT), lambda i, j: (i, 0, 0)),
        ],
        out_shape=[
            jax.ShapeDtypeStruct((_B, _T_IN, _FEAT), jnp.float32),
            jax.ShapeDtypeStruct((_B, _T_FC, _FEAT), jnp.float32),
        ],
    )(input, emb)


def kernel(input, tables):
    idx = input[:, 0, _N_NUM:]
    flat_idx = idx + (jnp.arange(_NUM_EMB, dtype=jnp.int32) * _VOCAB)[None, :]
    idx_hbm = flat_idx.reshape(_NW, _NCHUNK, _CHUNK)
    table_flat = tables.reshape(_NUM_EMB * _VOCAB, _EMB_DIM)
    emb = _make_sc_gather()(table_flat, idx_hbm).reshape(_B, _EMB_FEAT)
    past, fut = _assemble(input, emb)
    return (past, fut)


# speedup vs baseline: 1.5588x; 1.0509x over previous
import functools

import jax
import jax.numpy as jnp
from jax import lax
from jax.experimental import pallas as pl
from jax.experimental.pallas import tpu as pltpu
from jax.experimental.pallas import tpu_sc as plsc

_NUM_EMB = 10
_VOCAB = 100000
_EMB_DIM = 32
_N_NUM = 8
_T_IN = 168
_T_FC = 24
_T = _T_IN + _T_FC
_B = 1024
_EMB_FEAT = _NUM_EMB * _EMB_DIM
_FEAT = _N_NUM + _EMB_FEAT

_NC, _NS = 2, 16
_NW = _NC * _NS
_ROWS = _B * _NUM_EMB
_ROWS_PER_W = _ROWS // _NW
_CHUNK = 64
_NCHUNK = _ROWS_PER_W // _CHUNK


def _sc_gather_body(table_hbm, idx_hbm, out_hbm, idx_v, rows_v, sem):
    wid = lax.axis_index("s") * _NC + lax.axis_index("c")
    pltpu.sync_copy(idx_hbm.at[wid], idx_v)
    copies = [
        pltpu.async_copy(table_hbm.at[idx_v.at[j]], rows_v.at[j], sem)
        for j in range(_NCHUNK)
    ]
    for c in copies:
        c.wait()
    pltpu.sync_copy(rows_v, out_hbm.at[wid])


@functools.cache
def _make_sc_gather():
    return pl.kernel(
        _sc_gather_body,
        out_type=jax.ShapeDtypeStruct((_NW, _NCHUNK, _CHUNK, _EMB_DIM),
                                      jnp.float32),
        mesh=plsc.VectorSubcoreMesh(core_axis_name="c", subcore_axis_name="s",
                                    num_cores=_NC, num_subcores=_NS),
        scratch_types=[
            pltpu.VMEM((_NCHUNK, _CHUNK), jnp.int32),
            pltpu.VMEM((_NCHUNK, _CHUNK, _EMB_DIM), jnp.float32),
            pltpu.SemaphoreType.DMA,
        ],
        compiler_params=pltpu.CompilerParams(use_tc_tiling_on_sc=False),
    )

_BB = 32
_TB = 24
_NTB = _T // _TB
_NPB = _T_IN // _TB


def _asm_body(in_ref, emb_ref, past_ref, fut_ref):
    emb = emb_ref[...]
    embb = jnp.broadcast_to(emb[:, None, :], (_BB, _TB, _EMB_FEAT))
    for c in range(_NTB):
        num = in_ref[:, c * _TB:(c + 1) * _TB, :_N_NUM].astype(jnp.float32)
        full = jnp.concatenate([num, embb], axis=2)
        if c < _NPB:
            past_ref[:, c * _TB:(c + 1) * _TB, :] = full
        else:
            fut_ref[...] = full


def _assemble(input, emb):
    return pl.pallas_call(
        _asm_body,
        grid=(_B // _BB,),
        in_specs=[
            pl.BlockSpec((_BB, _T, _N_NUM + _NUM_EMB), lambda i: (i, 0, 0)),
            pl.BlockSpec((_BB, _EMB_FEAT), lambda i: (i, 0)),
        ],
        out_specs=[
            pl.BlockSpec((_BB, _T_IN, _FEAT), lambda i: (i, 0, 0)),
            pl.BlockSpec((_BB, _T_FC, _FEAT), lambda i: (i, 0, 0)),
        ],
        out_shape=[
            jax.ShapeDtypeStruct((_B, _T_IN, _FEAT), jnp.float32),
            jax.ShapeDtypeStruct((_B, _T_FC, _FEAT), jnp.float32),
        ],
    )(input, emb)


def kernel(input, tables):
    idx = input[:, 0, _N_NUM:]
    flat_idx = idx + (jnp.arange(_NUM_EMB, dtype=jnp.int32) * _VOCAB)[None, :]
    idx_hbm = flat_idx.reshape(_NW, _NCHUNK, _CHUNK)
    table_flat = tables.reshape(_NUM_EMB * _VOCAB, _EMB_DIM)
    emb = _make_sc_gather()(table_flat, idx_hbm).reshape(_B, _EMB_FEAT)
    past, fut = _assemble(input, emb)
    return (past, fut)


# speedup vs baseline: 1.5953x; 1.0234x over previous
import functools

import jax
import jax.numpy as jnp
from jax import lax
from jax.experimental import pallas as pl
from jax.experimental.pallas import tpu as pltpu
from jax.experimental.pallas import tpu_sc as plsc

_NUM_EMB = 10
_VOCAB = 100000
_EMB_DIM = 32
_N_NUM = 8
_T_IN = 168
_T_FC = 24
_T = _T_IN + _T_FC
_B = 1024
_EMB_FEAT = _NUM_EMB * _EMB_DIM
_FEAT = _N_NUM + _EMB_FEAT

_NC, _NS = 2, 16
_NW = _NC * _NS
_ROWS = _B * _NUM_EMB
_ROWS_PER_W = _ROWS // _NW
_CHUNK = 64
_NCHUNK = _ROWS_PER_W // _CHUNK


def _sc_gather_body(table_hbm, idx_hbm, out_hbm, idx_v, rows_v, sem):
    wid = lax.axis_index("s") * _NC + lax.axis_index("c")
    pltpu.sync_copy(idx_hbm.at[wid], idx_v)
    copies = [
        pltpu.async_copy(table_hbm.at[idx_v.at[j]], rows_v.at[j], sem)
        for j in range(_NCHUNK)
    ]
    for c in copies:
        c.wait()
    pltpu.sync_copy(rows_v, out_hbm.at[wid])


@functools.cache
def _make_sc_gather():
    return pl.kernel(
        _sc_gather_body,
        out_type=jax.ShapeDtypeStruct((_NW, _NCHUNK, _CHUNK, _EMB_DIM),
                                      jnp.float32),
        mesh=plsc.VectorSubcoreMesh(core_axis_name="c", subcore_axis_name="s",
                                    num_cores=_NC, num_subcores=_NS),
        scratch_types=[
            pltpu.VMEM((_NCHUNK, _CHUNK), jnp.int32),
            pltpu.VMEM((_NCHUNK, _CHUNK, _EMB_DIM), jnp.float32),
            pltpu.SemaphoreType.DMA,
        ],
        compiler_params=pltpu.CompilerParams(use_tc_tiling_on_sc=False),
    )

_BB = 64
_TB = 24
_NTB = _T // _TB
_NPB = _T_IN // _TB


def _asm_body(in_ref, emb_ref, past_ref, fut_ref):
    emb = emb_ref[...]
    embb = jnp.broadcast_to(emb[:, None, :], (_BB, _TB, _EMB_FEAT))
    for c in range(_NTB):
        num = in_ref[:, c * _TB:(c + 1) * _TB, :_N_NUM].astype(jnp.float32)
        full = jnp.concatenate([num, embb], axis=2)
        if c < _NPB:
            past_ref[:, c * _TB:(c + 1) * _TB, :] = full
        else:
            fut_ref[...] = full


def _assemble(input, emb):
    return pl.pallas_call(
        _asm_body,
        grid=(_B // _BB,),
        in_specs=[
            pl.BlockSpec((_BB, _T, _N_NUM + _NUM_EMB), lambda i: (i, 0, 0)),
            pl.BlockSpec((_BB, _EMB_FEAT), lambda i: (i, 0)),
        ],
        out_specs=[
            pl.BlockSpec((_BB, _T_IN, _FEAT), lambda i: (i, 0, 0)),
            pl.BlockSpec((_BB, _T_FC, _FEAT), lambda i: (i, 0, 0)),
        ],
        out_shape=[
            jax.ShapeDtypeStruct((_B, _T_IN, _FEAT), jnp.float32),
            jax.ShapeDtypeStruct((_B, _T_FC, _FEAT), jnp.float32),
        ],
        compiler_params=pltpu.CompilerParams(vmem_limit_bytes=120 * 1024 * 1024),
    )(input, emb)


def kernel(input, tables):
    idx = input[:, 0, _N_NUM:]
    flat_idx = idx + (jnp.arange(_NUM_EMB, dtype=jnp.int32) * _VOCAB)[None, :]
    idx_hbm = flat_idx.reshape(_NW, _NCHUNK, _CHUNK)
    table_flat = tables.reshape(_NUM_EMB * _VOCAB, _EMB_DIM)
    emb = _make_sc_gather()(table_flat, idx_hbm).reshape(_B, _EMB_FEAT)
    past, fut = _assemble(input, emb)
    return (past, fut)
